# baseline (speedup 1.0000x reference)
import jax
import jax.numpy as jnp
from jax.experimental import pallas as pl
from jax.experimental.pallas import tpu as pltpu

_B, _C, _H, _W = 4, 96, 512, 512
_BH = 64
_BW = 256
_TOTAL = _B * _H * _W
_NF = 1.0 / _C


def _body(x_ref, loss_ref, dist_ref, acc_ref):
    step = (pl.program_id(0) * pl.num_programs(1) + pl.program_id(1)) * pl.num_programs(2) + pl.program_id(2)
    nsteps = pl.num_programs(0) * pl.num_programs(1) * pl.num_programs(2)

    @pl.when(step == 0)
    def _init():
        acc_ref[...] = jnp.zeros_like(acc_ref)

    x = x_ref[0]
    idx = jnp.argmax(x, axis=0).astype(jnp.int32)
    classes = jax.lax.broadcasted_iota(jnp.int32, (_C, _BH, _BW), 0)
    onehot = (idx[None, :, :] == classes).astype(jnp.float32)
    part = jnp.sum(onehot.reshape(_C, _BH // 8, 8, _BW), axis=1)
    acc_ref[:, :, pl.ds(pl.program_id(2) * _BW, _BW)] += part

    @pl.when(step == nsteps - 1)
    def _fin():
        hist = jnp.sum(acc_ref[...], axis=(1, 2), keepdims=True)[:, 0, :]
        dist = hist * (1.0 / _TOTAL)
        dist_ref[...] = dist
        z = (dist - _NF) * (1.0 / (1.0 - _NF))
        loss_ref[0, 0] = jnp.sqrt(jnp.sum(z * z))


def kernel(generated_masks):
    loss2d, dist2d = pl.pallas_call(
        _body,
        grid=(_B, _H // _BH, _W // _BW),
        in_specs=[
            pl.BlockSpec((1, _C, _BH, _BW), lambda b, h, w: (b, 0, h, w)),
        ],
        out_specs=[
            pl.BlockSpec(memory_space=pltpu.SMEM),
            pl.BlockSpec((_C, 1), lambda b, h, w: (0, 0)),
        ],
        out_shape=[
            jax.ShapeDtypeStruct((1, 1), jnp.float32),
            jax.ShapeDtypeStruct((_C, 1), jnp.float32),
        ],
        scratch_shapes=[pltpu.VMEM((_C, 8, _W), jnp.float32)],
    )(generated_masks)
    return (loss2d[0, 0], dist2d[:, 0])


# speedup vs baseline: 1.0701x; 1.0701x over previous
import jax
import jax.numpy as jnp
from jax.experimental import pallas as pl
from jax.experimental.pallas import tpu as pltpu

_B, _C, _H, _W = 4, 96, 512, 512
_BH = 64
_TOTAL = _B * _H * _W
_NF = 1.0 / _C


def _body(x_ref, loss_ref, dist_ref, acc_ref):
    step = pl.program_id(0)
    nsteps = pl.num_programs(0)

    @pl.when(step == 0)
    def _init():
        acc_ref[...] = jnp.zeros_like(acc_ref)

    x = x_ref[0]
    idx = jnp.argmax(x, axis=0).astype(jnp.int32)
    classes = jax.lax.broadcasted_iota(jnp.int32, (_C, _BH, _W), 0)
    onehot = (idx[None, :, :] == classes).astype(jnp.float32)
    part = jnp.sum(onehot.reshape(_C, _BH // 8, 8, _W), axis=1)
    acc_ref[...] += part

    @pl.when(step == nsteps - 1)
    def _fin():
        hist = jnp.sum(acc_ref[...], axis=(1, 2), keepdims=True)[:, 0, :]
        dist = hist * (1.0 / _TOTAL)
        dist_ref[...] = dist
        z = (dist - _NF) * (1.0 / (1.0 - _NF))
        loss_ref[0, 0] = jnp.sqrt(jnp.sum(z * z))


def kernel(generated_masks):
    loss2d, dist2d = pl.pallas_call(
        _body,
        grid=(_B * _H // _BH,),
        in_specs=[
            pl.BlockSpec((1, _C, _BH, _W), lambda i: (i // (_H // _BH), 0, i % (_H // _BH), 0)),
        ],
        out_specs=[
            pl.BlockSpec(memory_space=pltpu.SMEM),
            pl.BlockSpec((_C, 1), lambda i: (0, 0)),
        ],
        out_shape=[
            jax.ShapeDtypeStruct((1, 1), jnp.float32),
            jax.ShapeDtypeStruct((_C, 1), jnp.float32),
        ],
        scratch_shapes=[pltpu.VMEM((_C, 8, _W), jnp.float32)],
    )(generated_masks)
    return (loss2d[0, 0], dist2d[:, 0])


# speedup vs baseline: 1.0825x; 1.0116x over previous
import jax
import jax.numpy as jnp
from jax.experimental import pallas as pl
from jax.experimental.pallas import tpu as pltpu

_B, _C, _H, _W = 4, 96, 512, 512
_BH = 64
_TOTAL = _B * _H * _W
_NF = 1.0 / _C


def _body(x_ref, loss_ref, dist_ref, acc_ref):
    step = pl.program_id(0) * pl.num_programs(1) + pl.program_id(1)
    nsteps = pl.num_programs(0) * pl.num_programs(1)

    @pl.when(step == 0)
    def _init():
        acc_ref[...] = jnp.zeros_like(acc_ref)

    x = x_ref[0]
    idx = jnp.argmax(x, axis=0).astype(jnp.int32)
    classes = jax.lax.broadcasted_iota(jnp.int32, (_C, _BH, _W), 0)
    onehot = (idx[None, :, :] == classes).astype(jnp.float32)
    part = jnp.sum(onehot.reshape(_C, _BH // 8, 8, _W), axis=1)
    acc_ref[...] += part

    @pl.when(step == nsteps - 1)
    def _fin():
        hist = jnp.sum(acc_ref[...], axis=(1, 2), keepdims=True)[:, 0, :]
        dist = hist * (1.0 / _TOTAL)
        dist_ref[...] = dist
        z = (dist - _NF) * (1.0 / (1.0 - _NF))
        loss_ref[0, 0] = jnp.sqrt(jnp.sum(z * z))


def kernel(generated_masks):
    loss2d, dist2d = pl.pallas_call(
        _body,
        grid=(_B, _H // _BH),
        in_specs=[
            pl.BlockSpec((1, _C, _BH, _W), lambda b, h: (b, 0, h, 0)),
        ],
        out_specs=[
            pl.BlockSpec(memory_space=pltpu.SMEM),
            pl.BlockSpec((_C, 1), lambda b, h: (0, 0)),
        ],
        out_shape=[
            jax.ShapeDtypeStruct((1, 1), jnp.float32),
            jax.ShapeDtypeStruct((_C, 1), jnp.float32),
        ],
        scratch_shapes=[pltpu.VMEM((_C, 8, _W), jnp.float32)],
    )(generated_masks)
    return (loss2d[0, 0], dist2d[:, 0])
